# baseline (speedup 1.0000x reference)
import functools

import jax
import jax.numpy as jnp
from jax.experimental import pallas as pl
from jax.experimental.pallas import tpu as pltpu

_N_HEADS = 8
_DH = 64
_KP = 4


def _blocksum64(p, tb):
    return jnp.sum(p.reshape(_N_HEADS, _DH, tb), axis=1)


def _bcast64(a, tb):
    return jnp.broadcast_to(a[:, None, :], (_N_HEADS, _DH, tb)).reshape(
        _N_HEADS * _DH, tb)


def _dotT(w, x):
    return jax.lax.dot_general(
        w, x, (((0,), (0,)), ((), ())), preferred_element_type=jnp.float32)


def _ffn_kernel(nf_ref, x_ref, ctx_ref, wN_ref,
                Wq_ref, bq_ref, Wk_ref, bk_ref, Wv_ref, bv_ref,
                pq_ref, A2T_ref, B2_ref, Wup_ref, bup_ref,
                Wdn_ref, bdn_ref, out_ref, k_scr, v_scr, nfT_scr):
    K, tb, D = nf_ref.shape
    n_pat = pq_ref.shape[0]
    f32 = jnp.float32

    i0 = jax.lax.broadcasted_iota(jnp.int32, (tb, tb), 0)
    i1 = jax.lax.broadcasted_iota(jnp.int32, (tb, tb), 1)
    eye = jnp.where(i0 == i1, 1.0, 0.0).astype(f32)

    def _tr(m):
        return _dotT(m, eye)

    Wq = Wq_ref[:]
    Wk = Wk_ref[:]
    Wv = Wv_ref[:]
    bq = bq_ref[:]
    bk = bk_ref[:]
    bv = bv_ref[:]

    for i in range(K):
        nf_i = _tr(nf_ref[i])
        nfT_scr[i] = nf_i
        k_scr[i] = _dotT(Wk, nf_i) + bk
        v_scr[i] = jax.nn.sigmoid(_dotT(Wv, nf_i) + bv)

    agg = jnp.zeros((D, tb), dtype=f32)
    wN = wN_ref[:]
    inv_sqrt_dh = 1.0 / (_DH ** 0.5)
    for i in range(K):
        nf_i = nfT_scr[i]
        q_i = _dotT(Wq, nf_i) + bq
        s_list = []
        for j in range(K):
            s_list.append(_blocksum64(q_i * k_scr[j], tb) * inv_sqrt_dh)
        m = functools.reduce(jnp.maximum, s_list)
        e_list = [jnp.exp(s - m) for s in s_list]
        den = functools.reduce(jnp.add, e_list)
        inv_den = 1.0 / den
        g_i = jnp.zeros((D, tb), dtype=f32)
        for j in range(K):
            g_i = g_i + _bcast64(e_list[j] * inv_den, tb) * v_scr[j]
        agg = agg + wN[i:i + 1, :] * nf_i * g_i

    pq = pq_ref[:]
    ctx_sc = jax.lax.dot_general(
        pq, ctx_ref[:], (((1,), (1,)), ((), ())), preferred_element_type=f32)
    scores = (jnp.dot(pq, agg, preferred_element_type=f32)
              * (0.5 / (D ** 0.5))
              + ctx_sc * 0.5)

    row_idx = jax.lax.broadcasted_iota(jnp.int32, (n_pat, tb), 0)
    work = scores
    sels = []
    vals = []
    for r in range(_KP):
        m_r = jnp.max(work, axis=0, keepdims=True)
        cand = jnp.where(work == m_r, row_idx, n_pat)
        first = jnp.min(cand, axis=0, keepdims=True)
        sel = (row_idx == first).astype(f32)
        sels.append(sel)
        vals.append(m_r)
        work = work - sel * 1e30
    e_vals = [jnp.exp(v - vals[0]) for v in vals]
    den_v = functools.reduce(jnp.add, e_vals)
    inv_den_v = 1.0 / den_v
    denseW = functools.reduce(
        jnp.add, [sels[r] * (e_vals[r] * inv_den_v) for r in range(_KP)])

    combT = _tr(x_ref[:]) + agg
    h_mid = jnp.dot(A2T_ref[:], combT, preferred_element_type=f32)
    nr = h_mid.shape[0]
    rank = nr // n_pat
    wexp = jnp.broadcast_to(denseW[:, None, :], (n_pat, rank, tb)).reshape(
        nr, tb)
    h_pat = _dotT(B2_ref[:], wexp * h_mid)
    h_base = _dotT(Wup_ref[:], combT) + bup_ref[:]
    h = 0.1 * h_base + 0.9 * h_pat
    h = 0.5 * h * (1.0 + jax.lax.erf(h * (2.0 ** -0.5)))
    out_ref[:] = _dotT(h, Wdn_ref[:]) + bdn_ref[:]


def kernel(x, selected_neurons, topk_neuron_weights, context, Wq, bq, Wk, bk,
           Wv, bv, pattern_queries, pattern_up_A, pattern_up_B, W_up, b_up,
           W_down, b_down):
    B, S, K, D = selected_neurons.shape
    T = B * S
    n_pat, _, rank = pattern_up_A.shape
    d_ff = pattern_up_B.shape[-1]
    TB = 256
    f32 = jnp.float32

    nf3 = selected_neurons.reshape(T, K, D).transpose(1, 0, 2)
    x2 = x.reshape(T, D)
    ctx2 = context.reshape(T, D)
    wN = topk_neuron_weights.reshape(T, K).T
    A2T = pattern_up_A.transpose(0, 2, 1).reshape(n_pat * rank, D)
    B2 = pattern_up_B.reshape(n_pat * rank, d_ff)

    grid = (T // TB,)
    tok = lambda j: (0, j)
    rowtok = lambda j: (j, 0)
    full2 = lambda j: (0, 0)
    full3 = lambda j: (0, 0, j)

    out = pl.pallas_call(
        _ffn_kernel,
        grid=grid,
        in_specs=[
            pl.BlockSpec((K, TB, D), lambda j: (0, j, 0)),
            pl.BlockSpec((TB, D), rowtok),
            pl.BlockSpec((TB, D), rowtok),
            pl.BlockSpec((K, TB), tok),
            pl.BlockSpec((D, D), full2),
            pl.BlockSpec((D, 1), full2),
            pl.BlockSpec((D, D), full2),
            pl.BlockSpec((D, 1), full2),
            pl.BlockSpec((D, D), full2),
            pl.BlockSpec((D, 1), full2),
            pl.BlockSpec((n_pat, D), full2),
            pl.BlockSpec((n_pat * rank, D), full2),
            pl.BlockSpec((n_pat * rank, d_ff), full2),
            pl.BlockSpec((D, d_ff), full2),
            pl.BlockSpec((d_ff, 1), full2),
            pl.BlockSpec((d_ff, D), full2),
            pl.BlockSpec((1, D), full2),
        ],
        out_specs=pl.BlockSpec((TB, D), rowtok),
        out_shape=jax.ShapeDtypeStruct((T, D), f32),
        scratch_shapes=[
            pltpu.VMEM((K, D, TB), f32),
            pltpu.VMEM((K, D, TB), f32),
            pltpu.VMEM((K, D, TB), f32),
        ],
    )(nf3, x2, ctx2, wN, Wq, bq[:, None], Wk, bk[:, None], Wv,
      bv[:, None], pattern_queries, A2T, B2, W_up, b_up[:, None],
      W_down, b_down[None, :])

    return out.reshape(B, S, D)


# speedup vs baseline: 1.0023x; 1.0023x over previous
import functools

import jax
import jax.numpy as jnp
from jax.experimental import pallas as pl
from jax.experimental.pallas import tpu as pltpu

_N_HEADS = 8
_DH = 64
_KP = 4


def _blocksum64(p, tb):
    return jnp.sum(p.reshape(_N_HEADS, _DH, tb), axis=1)


def _bcast64(a, tb):
    return jnp.broadcast_to(a[:, None, :], (_N_HEADS, _DH, tb)).reshape(
        _N_HEADS * _DH, tb)


def _dotT(w, x):
    return jax.lax.dot_general(
        w, x, (((0,), (0,)), ((), ())), preferred_element_type=jnp.float32)


def _ffn_kernel(nf_ref, x_ref, ctx_ref, wN_ref,
                Wq_ref, bq_ref, Wk_ref, bk_ref, Wv_ref, bv_ref,
                pq_ref, A2T_ref, B2_ref, Wup_ref, bup_ref,
                Wdn_ref, bdn_ref, out_ref, k_scr, v_scr, nfT_scr):
    tb = nf_ref.shape[0]
    D = x_ref.shape[1]
    K = nf_ref.shape[1] // D
    n_pat = pq_ref.shape[0]
    f32 = jnp.float32

    i0 = jax.lax.broadcasted_iota(jnp.int32, (tb, tb), 0)
    i1 = jax.lax.broadcasted_iota(jnp.int32, (tb, tb), 1)
    eye = jnp.where(i0 == i1, 1.0, 0.0).astype(f32)

    def _tr(m):
        return _dotT(m, eye)

    Wq = Wq_ref[:]
    Wk = Wk_ref[:]
    Wv = Wv_ref[:]
    bq = bq_ref[:]
    bk = bk_ref[:]
    bv = bv_ref[:]

    for i in range(K):
        nf_i = _tr(nf_ref[:, i * D:(i + 1) * D])
        nfT_scr[i] = nf_i
        k_scr[i] = _dotT(Wk, nf_i) + bk
        v_scr[i] = jax.nn.sigmoid(_dotT(Wv, nf_i) + bv)

    agg = jnp.zeros((D, tb), dtype=f32)
    wN = wN_ref[:]
    inv_sqrt_dh = 1.0 / (_DH ** 0.5)
    for i in range(K):
        nf_i = nfT_scr[i]
        q_i = _dotT(Wq, nf_i) + bq
        s_list = []
        for j in range(K):
            s_list.append(_blocksum64(q_i * k_scr[j], tb) * inv_sqrt_dh)
        m = functools.reduce(jnp.maximum, s_list)
        e_list = [jnp.exp(s - m) for s in s_list]
        den = functools.reduce(jnp.add, e_list)
        inv_den = 1.0 / den
        g_i = jnp.zeros((D, tb), dtype=f32)
        for j in range(K):
            g_i = g_i + _bcast64(e_list[j] * inv_den, tb) * v_scr[j]
        agg = agg + wN[i:i + 1, :] * nf_i * g_i

    pq = pq_ref[:]
    ctx_sc = jax.lax.dot_general(
        pq, ctx_ref[:], (((1,), (1,)), ((), ())), preferred_element_type=f32)
    scores = (jnp.dot(pq, agg, preferred_element_type=f32)
              * (0.5 / (D ** 0.5))
              + ctx_sc * 0.5)

    row_idx = jax.lax.broadcasted_iota(jnp.int32, (n_pat, tb), 0)
    work = scores
    sels = []
    vals = []
    for r in range(_KP):
        m_r = jnp.max(work, axis=0, keepdims=True)
        cand = jnp.where(work == m_r, row_idx, n_pat)
        first = jnp.min(cand, axis=0, keepdims=True)
        sel = (row_idx == first).astype(f32)
        sels.append(sel)
        vals.append(m_r)
        work = work - sel * 1e30
    e_vals = [jnp.exp(v - vals[0]) for v in vals]
    den_v = functools.reduce(jnp.add, e_vals)
    inv_den_v = 1.0 / den_v
    denseW = functools.reduce(
        jnp.add, [sels[r] * (e_vals[r] * inv_den_v) for r in range(_KP)])

    combT = _tr(x_ref[:]) + agg
    h_mid = jnp.dot(A2T_ref[:], combT, preferred_element_type=f32)
    nr = h_mid.shape[0]
    rank = nr // n_pat
    wexp = jnp.broadcast_to(denseW[:, None, :], (n_pat, rank, tb)).reshape(
        nr, tb)
    h_pat = _dotT(B2_ref[:], wexp * h_mid)
    h_base = _dotT(Wup_ref[:], combT) + bup_ref[:]
    h = 0.1 * h_base + 0.9 * h_pat
    h = 0.5 * h * (1.0 + jax.lax.erf(h * (2.0 ** -0.5)))
    out_ref[:] = _dotT(h, Wdn_ref[:]) + bdn_ref[:]


def kernel(x, selected_neurons, topk_neuron_weights, context, Wq, bq, Wk, bk,
           Wv, bv, pattern_queries, pattern_up_A, pattern_up_B, W_up, b_up,
           W_down, b_down):
    B, S, K, D = selected_neurons.shape
    T = B * S
    n_pat, _, rank = pattern_up_A.shape
    d_ff = pattern_up_B.shape[-1]
    TB = 256
    f32 = jnp.float32

    nf2 = selected_neurons.reshape(T, K * D)
    x2 = x.reshape(T, D)
    ctx2 = context.reshape(T, D)
    wN = topk_neuron_weights.reshape(T, K).T
    A2T = pattern_up_A.transpose(0, 2, 1).reshape(n_pat * rank, D)
    B2 = pattern_up_B.reshape(n_pat * rank, d_ff)

    grid = (T // TB,)
    tok = lambda j: (0, j)
    rowtok = lambda j: (j, 0)
    full2 = lambda j: (0, 0)
    full3 = lambda j: (0, 0, j)

    out = pl.pallas_call(
        _ffn_kernel,
        grid=grid,
        in_specs=[
            pl.BlockSpec((TB, K * D), rowtok),
            pl.BlockSpec((TB, D), rowtok),
            pl.BlockSpec((TB, D), rowtok),
            pl.BlockSpec((K, TB), tok),
            pl.BlockSpec((D, D), full2),
            pl.BlockSpec((D, 1), full2),
            pl.BlockSpec((D, D), full2),
            pl.BlockSpec((D, 1), full2),
            pl.BlockSpec((D, D), full2),
            pl.BlockSpec((D, 1), full2),
            pl.BlockSpec((n_pat, D), full2),
            pl.BlockSpec((n_pat * rank, D), full2),
            pl.BlockSpec((n_pat * rank, d_ff), full2),
            pl.BlockSpec((D, d_ff), full2),
            pl.BlockSpec((d_ff, 1), full2),
            pl.BlockSpec((d_ff, D), full2),
            pl.BlockSpec((1, D), full2),
        ],
        out_specs=pl.BlockSpec((TB, D), rowtok),
        out_shape=jax.ShapeDtypeStruct((T, D), f32),
        scratch_shapes=[
            pltpu.VMEM((K, D, TB), f32),
            pltpu.VMEM((K, D, TB), f32),
            pltpu.VMEM((K, D, TB), f32),
        ],
    )(nf2, x2, ctx2, wN, Wq, bq[:, None], Wk, bk[:, None], Wv,
      bv[:, None], pattern_queries, A2T, B2, W_up, b_up[:, None],
      W_down, b_down[None, :])

    return out.reshape(B, S, D)


# speedup vs baseline: 1.0454x; 1.0430x over previous
import functools

import jax
import jax.numpy as jnp
from jax.experimental import pallas as pl
from jax.experimental.pallas import tpu as pltpu

_N_HEADS = 8
_DH = 64
_KP = 4


def _blocksum64(p, tb):
    return jnp.sum(p.reshape(_N_HEADS, _DH, tb), axis=1)


def _bcast64(a, tb):
    return jnp.broadcast_to(a[:, None, :], (_N_HEADS, _DH, tb)).reshape(
        _N_HEADS * _DH, tb)


def _dotT(w, x):
    return jax.lax.dot_general(
        w, x, (((0,), (0,)), ((), ())), preferred_element_type=jnp.float32)


def _ffn_kernel(nf_ref, x_ref, ctx_ref, wN_ref,
                Wq_ref, bq_ref, Wk_ref, bk_ref, Wv_ref, bv_ref,
                pq_ref, A2T_ref, B2_ref, Wup_ref, bup_ref,
                Wdn_ref, bdn_ref, out_ref, k_scr, v_scr, nfT_scr):
    tb = nf_ref.shape[0]
    D = x_ref.shape[1]
    K = nf_ref.shape[1] // D
    n_pat = pq_ref.shape[0]
    f32 = jnp.float32

    def _tr(m):
        return jnp.transpose(m)

    Wq = Wq_ref[:]
    Wk = Wk_ref[:]
    Wv = Wv_ref[:]
    bq = bq_ref[:]
    bk = bk_ref[:]
    bv = bv_ref[:]

    for i in range(K):
        nf_i = _tr(nf_ref[:, i * D:(i + 1) * D])
        nfT_scr[i] = nf_i
        k_scr[i] = _dotT(Wk, nf_i) + bk
        v_scr[i] = jax.nn.sigmoid(_dotT(Wv, nf_i) + bv)

    agg = jnp.zeros((D, tb), dtype=f32)
    wN = wN_ref[:]
    inv_sqrt_dh = 1.0 / (_DH ** 0.5)
    for i in range(K):
        nf_i = nfT_scr[i]
        q_i = _dotT(Wq, nf_i) + bq
        s_list = []
        for j in range(K):
            s_list.append(_blocksum64(q_i * k_scr[j], tb) * inv_sqrt_dh)
        m = functools.reduce(jnp.maximum, s_list)
        e_list = [jnp.exp(s - m) for s in s_list]
        den = functools.reduce(jnp.add, e_list)
        inv_den = 1.0 / den
        g_i = jnp.zeros((D, tb), dtype=f32)
        for j in range(K):
            g_i = g_i + _bcast64(e_list[j] * inv_den, tb) * v_scr[j]
        agg = agg + wN[i:i + 1, :] * nf_i * g_i

    pq = pq_ref[:]
    ctx_sc = jax.lax.dot_general(
        pq, ctx_ref[:], (((1,), (1,)), ((), ())), preferred_element_type=f32)
    scores = (jnp.dot(pq, agg, preferred_element_type=f32)
              * (0.5 / (D ** 0.5))
              + ctx_sc * 0.5)

    row_idx = jax.lax.broadcasted_iota(jnp.int32, (n_pat, tb), 0)
    work = scores
    sels = []
    vals = []
    for r in range(_KP):
        m_r = jnp.max(work, axis=0, keepdims=True)
        cand = jnp.where(work == m_r, row_idx, n_pat)
        first = jnp.min(cand, axis=0, keepdims=True)
        sel = (row_idx == first).astype(f32)
        sels.append(sel)
        vals.append(m_r)
        work = work - sel * 1e30
    e_vals = [jnp.exp(v - vals[0]) for v in vals]
    den_v = functools.reduce(jnp.add, e_vals)
    inv_den_v = 1.0 / den_v
    denseW = functools.reduce(
        jnp.add, [sels[r] * (e_vals[r] * inv_den_v) for r in range(_KP)])

    combT = _tr(x_ref[:]) + agg
    h_mid = jnp.dot(A2T_ref[:], combT, preferred_element_type=f32)
    nr = h_mid.shape[0]
    rank = nr // n_pat
    wexp = jnp.broadcast_to(denseW[:, None, :], (n_pat, rank, tb)).reshape(
        nr, tb)
    h_pat = _dotT(B2_ref[:], wexp * h_mid)
    h_base = _dotT(Wup_ref[:], combT) + bup_ref[:]
    h = 0.1 * h_base + 0.9 * h_pat
    h = 0.5 * h * (1.0 + jax.lax.erf(h * (2.0 ** -0.5)))
    out_ref[:] = _dotT(h, Wdn_ref[:]) + bdn_ref[:]


def kernel(x, selected_neurons, topk_neuron_weights, context, Wq, bq, Wk, bk,
           Wv, bv, pattern_queries, pattern_up_A, pattern_up_B, W_up, b_up,
           W_down, b_down):
    B, S, K, D = selected_neurons.shape
    T = B * S
    n_pat, _, rank = pattern_up_A.shape
    d_ff = pattern_up_B.shape[-1]
    TB = 256
    f32 = jnp.float32

    nf2 = selected_neurons.reshape(T, K * D)
    x2 = x.reshape(T, D)
    ctx2 = context.reshape(T, D)
    wN = topk_neuron_weights.reshape(T, K).T
    A2T = pattern_up_A.transpose(0, 2, 1).reshape(n_pat * rank, D)
    B2 = pattern_up_B.reshape(n_pat * rank, d_ff)

    grid = (T // TB,)
    tok = lambda j: (0, j)
    rowtok = lambda j: (j, 0)
    full2 = lambda j: (0, 0)
    full3 = lambda j: (0, 0, j)

    out = pl.pallas_call(
        _ffn_kernel,
        grid=grid,
        in_specs=[
            pl.BlockSpec((TB, K * D), rowtok),
            pl.BlockSpec((TB, D), rowtok),
            pl.BlockSpec((TB, D), rowtok),
            pl.BlockSpec((K, TB), tok),
            pl.BlockSpec((D, D), full2),
            pl.BlockSpec((D, 1), full2),
            pl.BlockSpec((D, D), full2),
            pl.BlockSpec((D, 1), full2),
            pl.BlockSpec((D, D), full2),
            pl.BlockSpec((D, 1), full2),
            pl.BlockSpec((n_pat, D), full2),
            pl.BlockSpec((n_pat * rank, D), full2),
            pl.BlockSpec((n_pat * rank, d_ff), full2),
            pl.BlockSpec((D, d_ff), full2),
            pl.BlockSpec((d_ff, 1), full2),
            pl.BlockSpec((d_ff, D), full2),
            pl.BlockSpec((1, D), full2),
        ],
        out_specs=pl.BlockSpec((TB, D), rowtok),
        out_shape=jax.ShapeDtypeStruct((T, D), f32),
        scratch_shapes=[
            pltpu.VMEM((K, D, TB), f32),
            pltpu.VMEM((K, D, TB), f32),
            pltpu.VMEM((K, D, TB), f32),
        ],
    )(nf2, x2, ctx2, wN, Wq, bq[:, None], Wk, bk[:, None], Wv,
      bv[:, None], pattern_queries, A2T, B2, W_up, b_up[:, None],
      W_down, b_down[None, :])

    return out.reshape(B, S, D)
